# baseline (speedup 1.0000x reference)
import jax
import jax.numpy as jnp
from jax import lax
from jax.experimental import pallas as pl
from jax.experimental.pallas import tpu as pltpu
from jax.experimental.pallas import tpu_sc as plsc

N = 10001
D = 128
E = 320000
NP = 10112
NC = 2
NS = 16
NW = NC * NS
C = 80
SUP = 25
SUPE = SUP * C
EW = E // NW
NSUP = EW // SUPE
NPS = NP // NS
L = 16


def _sc_body(table, row4, col4, vals, zeros, part_out,
             acc, rowb, colb, valb, g0, g1, g2, sb0, sb1,
             gsem0, gsem1, gsem2, ssem0, ssem1, stsem):
    c = lax.axis_index("c")
    s = lax.axis_index("s")
    w = c * NS + s
    rbase = s * NPS
    half = c * NP

    pltpu.sync_copy(zeros.at[pl.ds(rbase, NPS)], acc.at[pl.ds(rbase, NPS)])
    plsc.subcore_barrier()

    def stage_ops(m, pty):
        yield row4.at[w, m], rowb.at[pty]
        yield col4.at[w, m], colb.at[pty]

    def stage_start(m, pty):
        for src, dst in stage_ops(m, pty):
            pltpu.async_copy(src, dst, stsem)

    def stage_wait(m, pty):
        for src, dst in stage_ops(m, pty):
            pltpu.make_async_copy(src, dst, stsem).wait()

    stage_start(0, 0)

    def sup(m, _):
        p = m % 2
        pltpu.sync_copy(vals.at[pl.ds(w * EW + m * SUPE, SUPE)],
                        valb.at[pl.ds(0, SUPE)])
        stage_wait(m, p)

        @pl.when(m < NSUP - 1)
        def _():
            stage_start(m + 1, 1 - p)

        gring = ((g0, gsem0), (g1, gsem1), (g2, gsem2))
        sring = ((sb0, ssem0), (sb1, ssem1))

        def g_start(k):
            buf, gsem = gring[k % 3]
            pltpu.async_copy(table.at[colb.at[p, k]], buf, gsem)

        def g_wait(k):
            buf, gsem = gring[k % 3]
            pltpu.make_async_copy(table.at[colb.at[p, k]], buf, gsem).wait()

        def s_start(k):
            buf, ssem = sring[k % 2]
            pltpu.async_copy(buf, acc.at[rowb.at[p, k]], ssem, add=True)

        def s_wait(k):
            buf, ssem = sring[k % 2]
            pltpu.make_async_copy(buf, acc.at[rowb.at[p, k]], ssem).wait()

        def scale(k):
            gb = gring[k % 3][0]
            sb = sring[k % 2][0]

            def edge(i, _):
                vv = jnp.broadcast_to(valb[pl.ds(k * C + i, L)][0], (L,))
                for j in range(D // (2 * L)):
                    xi = gb[i, pl.ds(L * j, L)]
                    a = lax.bitcast_convert_type(xi << 16, jnp.float32)
                    b = lax.bitcast_convert_type(
                        xi & jnp.int32(-65536), jnp.float32)
                    sb[i, pl.ds(2 * L * j, L)] = a * vv
                    sb[i, pl.ds(2 * L * j + L, L)] = b * vv
                return 0

            lax.fori_loop(0, C, edge, 0, unroll=4)

        g_start(0)
        g_start(1)
        for k in range(SUP):
            g_wait(k)
            if k >= 2:
                s_wait(k - 2)
            scale(k)
            s_start(k)
            if k + 2 < SUP:
                g_start(k + 2)
        s_wait(SUP - 2)
        s_wait(SUP - 1)
        return 0

    lax.fori_loop(0, NSUP, sup, 0)
    plsc.subcore_barrier()

    pltpu.sync_copy(acc.at[pl.ds(rbase, NPS)],
                    part_out.at[pl.ds(half + rbase, NPS)])


def _tc_merge_body(p0_ref, p1_ref, out_ref):
    out_ref[...] = p0_ref[...] + p1_ref[...]


def _tc_final_body(e0_ref, e1_ref, q0_ref, q1_ref, e1o_ref, e2_ref,
                   sum_ref):
    e1 = e1_ref[...]
    e2 = q0_ref[...] + q1_ref[...]
    e1o_ref[...] = e1
    e2_ref[...] = e2
    sum_ref[...] = e0_ref[...] + e1 + e2


_BLK = 632
_SPEC0 = pl.BlockSpec((_BLK, D), lambda i: (i, 0))
_SPEC1 = pl.BlockSpec((_BLK, D), lambda i: (i + NP // _BLK, 0))


def _sc_layer(table, row4, col4, vals, zeros):
    mesh = plsc.VectorSubcoreMesh(core_axis_name="c", subcore_axis_name="s")
    sc = pl.kernel(
        _sc_body,
        out_type=jax.ShapeDtypeStruct((NC * NP, D), jnp.float32),
        mesh=mesh,
        compiler_params=pltpu.CompilerParams(use_tc_tiling_on_sc=False),
        scratch_types=[
            pltpu.VMEM_SHARED((NP, D), jnp.float32),
            pltpu.VMEM((2, SUP, C), jnp.int32),
            pltpu.VMEM((2, SUP, C), jnp.int32),
            pltpu.VMEM((SUPE + L,), jnp.float32),
            pltpu.VMEM((C, D // 2), jnp.int32),
            pltpu.VMEM((C, D // 2), jnp.int32),
            pltpu.VMEM((C, D // 2), jnp.int32),
            pltpu.VMEM((C, D), jnp.float32),
            pltpu.VMEM((C, D), jnp.float32),
            pltpu.SemaphoreType.DMA,
            pltpu.SemaphoreType.DMA,
            pltpu.SemaphoreType.DMA,
            pltpu.SemaphoreType.DMA,
            pltpu.SemaphoreType.DMA,
            pltpu.SemaphoreType.DMA,
        ],
    )
    return sc(table, row4, col4, vals, zeros)


def _to_bf16_table(x):
    q = jnp.arange(D)
    perm = (q // 32) * 32 + (q % 2) * L + (q % 32) // 2
    tb = x.astype(jnp.bfloat16)[:, perm]
    return jax.lax.bitcast_convert_type(
        tb.reshape(x.shape[0], D // 2, 2), jnp.int32)


@jax.jit
def _run(emb, row4, col4, vals, zeros):
    p_parts = _sc_layer(_to_bf16_table(emb), row4, col4, vals, zeros)

    e1 = p_parts[:N] + p_parts[NP:NP + N]

    q_parts = _sc_layer(_to_bf16_table(e1), row4, col4, vals, zeros)

    e1o, e2, ssum = pl.pallas_call(
        _tc_final_body,
        grid=(pl.cdiv(N, _BLK),),
        in_specs=[_SPEC0, _SPEC0, _SPEC0, _SPEC1],
        out_specs=[_SPEC0, _SPEC0, _SPEC0],
        out_shape=(
            jax.ShapeDtypeStruct((N, D), jnp.float32),
            jax.ShapeDtypeStruct((N, D), jnp.float32),
            jax.ShapeDtypeStruct((N, D), jnp.float32),
        ),
    )(emb, e1, q_parts, q_parts)
    return e1o, e2, ssum


def kernel(edge_index, edge_values, item_emb):
    row = edge_index[0].astype(jnp.int32)
    col = edge_index[1].astype(jnp.int32)
    row4 = row.reshape(NW, NSUP, SUP, C)
    col4 = col.reshape(NW, NSUP, SUP, C)
    zeros = jnp.zeros((NP, D), jnp.float32)

    e1, e2, ssum = _run(item_emb, row4, col4, edge_values, zeros)
    return (ssum, item_emb, e1, e2)


# speedup vs baseline: 1.8176x; 1.8176x over previous
import jax
import jax.numpy as jnp
from jax import lax
from jax.experimental import pallas as pl
from jax.experimental.pallas import tpu as pltpu
from jax.experimental.pallas import tpu_sc as plsc

N = 10001
D = 128
E = 320000
NP = 10112
NC = 2
NS = 16
NW = NC * NS
C = 80
SUP = 25
SUPE = SUP * C
EW = E // NW
NSUP = EW // SUPE
NPS = NP // NS
L = 16


def _sc_body(table, row4, col4, vals, zeros, part_out,
             acc, rowb, colb, valb, g0, g1, g2,
             gsem0, gsem1, gsem2, ssem0, ssem1, ssem2, stsem):
    c = lax.axis_index("c")
    s = lax.axis_index("s")
    w = c * NS + s
    rbase = s * NPS
    half = c * NP

    pltpu.sync_copy(zeros.at[pl.ds(rbase, NPS)], acc.at[pl.ds(rbase, NPS)])
    plsc.subcore_barrier()

    def stage_ops(m, pty):
        yield row4.at[w, m], rowb.at[pty]
        yield col4.at[w, m], colb.at[pty]

    def stage_start(m, pty):
        for src, dst in stage_ops(m, pty):
            pltpu.async_copy(src, dst, stsem)

    def stage_wait(m, pty):
        for src, dst in stage_ops(m, pty):
            pltpu.make_async_copy(src, dst, stsem).wait()

    stage_start(0, 0)

    def sup(m, _):
        p = m % 2
        pltpu.sync_copy(vals.at[pl.ds(w * EW + m * SUPE, SUPE)],
                        valb.at[pl.ds(0, SUPE)])
        stage_wait(m, p)

        @pl.when(m < NSUP - 1)
        def _():
            stage_start(m + 1, 1 - p)

        ring = ((g0, gsem0, ssem0), (g1, gsem1, ssem1), (g2, gsem2, ssem2))

        def g_start(k):
            buf, gsem, _ = ring[k % 3]
            pltpu.async_copy(table.at[colb.at[p, k]], buf, gsem)

        def g_wait(k):
            buf, gsem, _ = ring[k % 3]
            pltpu.make_async_copy(table.at[colb.at[p, k]], buf, gsem).wait()

        def s_start(k):
            buf, _, ssem = ring[k % 3]
            pltpu.async_copy(buf, acc.at[rowb.at[p, k]], ssem, add=True)

        def s_wait(k):
            buf, _, ssem = ring[k % 3]
            pltpu.make_async_copy(buf, acc.at[rowb.at[p, k]], ssem).wait()

        def scale(k):
            buf = ring[k % 3][0]

            def edge(i, _):
                vv = jnp.broadcast_to(valb[pl.ds(k * C + i, L)][0], (L,))
                for j in range(D // L):
                    buf[i, pl.ds(j * L, L)] = buf[i, pl.ds(j * L, L)] * vv
                return 0

            lax.fori_loop(0, C, edge, 0, unroll=4)

        g_start(0)
        g_start(1)
        for k in range(SUP):
            g_wait(k)
            scale(k)
            s_start(k)
            if k + 2 < SUP:
                if k >= 1:
                    s_wait(k - 1)
                g_start(k + 2)
        s_wait(SUP - 3)
        s_wait(SUP - 2)
        s_wait(SUP - 1)
        return 0

    lax.fori_loop(0, NSUP, sup, 0)
    plsc.subcore_barrier()

    pltpu.sync_copy(acc.at[pl.ds(rbase, NPS)],
                    part_out.at[pl.ds(half + rbase, NPS)])


def _tc_final_body(e0_ref, e1_ref, q0_ref, q1_ref, e1o_ref, e2_ref,
                   sum_ref):
    e1 = e1_ref[...]
    e2 = q0_ref[...] + q1_ref[...]
    e1o_ref[...] = e1
    e2_ref[...] = e2
    sum_ref[...] = e0_ref[...] + e1 + e2


_BLK = 632
_SPEC0 = pl.BlockSpec((_BLK, D), lambda i: (i, 0))
_SPEC1 = pl.BlockSpec((_BLK, D), lambda i: (i + NP // _BLK, 0))


def _sc_layer(table, row4, col4, vals, zeros):
    mesh = plsc.VectorSubcoreMesh(core_axis_name="c", subcore_axis_name="s")
    sc = pl.kernel(
        _sc_body,
        out_type=jax.ShapeDtypeStruct((NC * NP, D), jnp.float32),
        mesh=mesh,
        scratch_types=[
            pltpu.VMEM_SHARED((NP, D), jnp.float32),
            pltpu.VMEM((2, SUP, C), jnp.int32),
            pltpu.VMEM((2, SUP, C), jnp.int32),
            pltpu.VMEM((SUPE + L,), jnp.float32),
            pltpu.VMEM((C, D), jnp.float32),
            pltpu.VMEM((C, D), jnp.float32),
            pltpu.VMEM((C, D), jnp.float32),
            pltpu.SemaphoreType.DMA,
            pltpu.SemaphoreType.DMA,
            pltpu.SemaphoreType.DMA,
            pltpu.SemaphoreType.DMA,
            pltpu.SemaphoreType.DMA,
            pltpu.SemaphoreType.DMA,
            pltpu.SemaphoreType.DMA,
        ],
    )
    return sc(table, row4, col4, vals, zeros)


@jax.jit
def _run(emb, row4, col4, vals, zeros):
    p_parts = _sc_layer(emb, row4, col4, vals, zeros)

    e1 = p_parts[:N] + p_parts[NP:NP + N]

    q_parts = _sc_layer(e1, row4, col4, vals, zeros)

    e1o, e2, ssum = pl.pallas_call(
        _tc_final_body,
        grid=(pl.cdiv(N, _BLK),),
        in_specs=[_SPEC0, _SPEC0, _SPEC0, _SPEC1],
        out_specs=[_SPEC0, _SPEC0, _SPEC0],
        out_shape=(
            jax.ShapeDtypeStruct((N, D), jnp.float32),
            jax.ShapeDtypeStruct((N, D), jnp.float32),
            jax.ShapeDtypeStruct((N, D), jnp.float32),
        ),
    )(emb, e1, q_parts, q_parts)
    return e1o, e2, ssum


def kernel(edge_index, edge_values, item_emb):
    row = edge_index[0].astype(jnp.int32)
    col = edge_index[1].astype(jnp.int32)
    row4 = row.reshape(NW, NSUP, SUP, C)
    col4 = col.reshape(NW, NSUP, SUP, C)
    zeros = jnp.zeros((NP, D), jnp.float32)

    e1, e2, ssum = _run(item_emb, row4, col4, edge_values, zeros)
    return (ssum, item_emb, e1, e2)
